# baseline (speedup 1.0000x reference)
import functools

import jax
import jax.numpy as jnp
from jax import lax
from jax.experimental import pallas as pl
from jax.experimental.pallas import tpu as pltpu
from jax.experimental.pallas import tpu_sc as plsc

_N = 100000
_E = 1600000
_IN_C = 32
_OUT_C = 64
_NET = 7
_AUG = 37
_YC = _NET * _OUT_C
_WROWS = 40

_ROW_BLK = 2048
_GRID = (_N + _ROW_BLK - 1) // _ROW_BLK

_NTILE = 16
_BLK = 400
_EPT = _E // _NTILE
_BLOCKS = _EPT // _BLK
_PAIRS = _BLOCKS // 2
_ACC_ROWS = _N + 8
_OPT = _N // _NTILE
_ZR = 125


def _y_body(x_ref, nt_ref, w_ref, y_ref):
    xb = x_ref[...]
    ntb = nt_ref[...]
    w1 = w_ref[:_IN_C, :]
    w2 = w_ref[_IN_C:, :]
    oh = (lax.broadcasted_iota(jnp.int32, (_ROW_BLK, _WROWS - _IN_C), 1)
          == ntb).astype(jnp.float32)
    acc = jnp.dot(xb, w1, preferred_element_type=jnp.float32)
    acc = acc + jnp.dot(oh, w2, preferred_element_type=jnp.float32)
    y_ref[...] = acc.astype(jnp.bfloat16)


_y_call = pl.pallas_call(
    _y_body,
    grid=(_GRID,),
    in_specs=[
        pl.BlockSpec((_ROW_BLK, _IN_C), lambda i: (i, 0)),
        pl.BlockSpec((_ROW_BLK, 1), lambda i: (i, 0)),
        pl.BlockSpec((_WROWS, _YC), lambda i: (0, 0)),
    ],
    out_specs=pl.BlockSpec((_ROW_BLK, _YC), lambda i: (i, 0)),
    out_shape=jax.ShapeDtypeStruct((_N, _YC), jnp.bfloat16),
)


@functools.partial(
    pl.kernel,
    out_type=jax.ShapeDtypeStruct((_N, _OUT_C), jnp.float32),
    mesh=plsc.VectorSubcoreMesh(core_axis_name="c", subcore_axis_name="s"),
    compiler_params=pltpu.CompilerParams(use_tc_tiling_on_sc=False,
                                         needs_layout_passes=False),
    scratch_types=[
        pltpu.VMEM((4, 1, _BLK), jnp.int32),
        pltpu.VMEM((4, 1, _BLK), jnp.int32),
        pltpu.VMEM((4, 1, _BLK), jnp.int32),
        pltpu.VMEM((4, 1, _BLK), jnp.int32),
        pltpu.VMEM((2, 1, _BLK, 32), jnp.bfloat16),
        pltpu.VMEM((_ZR, 32), jnp.bfloat16),
        pltpu.VMEM((_ZR, 32), jnp.bfloat16),
        pltpu.VMEM((_ZR, 32), jnp.float32),
        pltpu.VMEM_SHARED((_ACC_ROWS, 32), jnp.bfloat16),
        pltpu.SemaphoreType.DMA,
        pltpu.SemaphoreType.DMA,
        pltpu.SemaphoreType.DMA,
    ],
)
def _sc_call(y2, rowm, colm, typm, out, row_v, col_v, typ_v, idx_v, ybuf,
             zbuf, cbuf, fbuf, acc, sem_e, sem_g, sem_s):
    c = lax.axis_index("c")
    s = lax.axis_index("s")

    def zfill(i, carry):
        zbuf[i, :] = jnp.zeros((32,), jnp.bfloat16)
        return carry

    lax.fori_loop(0, _ZR, zfill, 0)

    def edges_start(e0, p):
        pltpu.async_copy(rowm.at[pl.ds(e0, _BLK)], row_v.at[p, 0], sem_e)
        pltpu.async_copy(colm.at[pl.ds(e0, _BLK)], col_v.at[p, 0], sem_e)
        pltpu.async_copy(typm.at[pl.ds(e0, _BLK)], typ_v.at[p, 0], sem_e)

    def edges_wait(p):
        pltpu.make_async_copy(rowm.at[pl.ds(0, _BLK)], row_v.at[p, 0],
                              sem_e).wait()
        pltpu.make_async_copy(colm.at[pl.ds(0, _BLK)], col_v.at[p, 0],
                              sem_e).wait()
        pltpu.make_async_copy(typm.at[pl.ds(0, _BLK)], typ_v.at[p, 0],
                              sem_e).wait()

    def idx_compute(q):
        for k in range(_BLK // 16):
            sl = pl.ds(k * 16, 16)
            idx_v[q, 0, sl] = (col_v[q, 0, sl] * 14
                               + typ_v[q, 0, sl] * 2 + c)

    def gather_start(q, p):
        pltpu.async_copy(y2.at[idx_v.at[q, 0]], ybuf.at[p, 0], sem_g)

    def gather_wait(q, p):
        pltpu.make_async_copy(y2.at[idx_v.at[q, 0]], ybuf.at[p, 0],
                              sem_g).wait()

    def scatter_start(q, p):
        pltpu.async_copy(ybuf.at[p, 0], acc.at[row_v.at[q, 0]], sem_s,
                         add=True)

    def scatter_wait(q, p):
        pltpu.make_async_copy(ybuf.at[p, 0], acc.at[row_v.at[q, 0]],
                              sem_s).wait()

    def zcp(z, carry):
        pltpu.sync_copy(zbuf, acc.at[pl.ds(s * _OPT + z * _ZR, _ZR), :])
        return carry

    lax.fori_loop(0, _OPT // _ZR, zcp, 0)
    plsc.subcore_barrier()

    base = s * _EPT

    for k in range(_BLK // 16):
        sl = pl.ds(k * 16, 16)
        idx_v[2, 0, sl] = jnp.zeros((16,), jnp.int32)
        idx_v[3, 0, sl] = jnp.zeros((16,), jnp.int32)
        row_v[2, 0, sl] = jnp.full((16,), _N, jnp.int32)
        row_v[3, 0, sl] = jnp.full((16,), _N, jnp.int32)
    gather_start(2, 0)
    gather_wait(2, 0)
    scatter_start(2, 0)
    gather_start(3, 1)
    edges_start(base, 0)
    edges_start(base + _BLK, 1)

    def pair_body(g, carry):
        b0 = base + 2 * g * _BLK
        q0 = 2 * (g % 2)
        q1 = q0 + 1
        qp = (q0 + 3) % 4
        edges_wait(q0)
        idx_compute(q0)
        gather_wait(qp, 1)
        scatter_start(qp, 1)
        scatter_wait((q0 + 2) % 4, 0)
        gather_start(q0, 0)
        @pl.when(g < _PAIRS - 1)
        def _():
            edges_start(b0 + 2 * _BLK, (q0 + 2) % 4)
        edges_wait(q1)
        idx_compute(q1)
        gather_wait(q0, 0)
        scatter_start(q0, 0)
        scatter_wait((q1 + 2) % 4, 1)
        gather_start(q1, 1)
        @pl.when(g < _PAIRS - 1)
        def _():
            edges_start(b0 + 3 * _BLK, (q1 + 2) % 4)
        return carry

    lax.fori_loop(0, _PAIRS, pair_body, 0)
    qlast = 2 * ((_PAIRS - 1) % 2) + 1
    gather_wait(qlast, 1)
    scatter_start(qlast, 1)
    scatter_wait((qlast + 3) % 4, 0)
    scatter_wait(qlast, 1)
    plsc.subcore_barrier()

    def drain(z, carry):
        r0 = s * _OPT + z * _ZR
        pltpu.sync_copy(acc.at[pl.ds(r0, _ZR), :], cbuf)

        def widen(r, carry2):
            a, b = plsc.unpack(cbuf[r, :], format=plsc.PackFormat.INTERLEAVED)
            fbuf[r, pl.ds(0, 16)] = a
            fbuf[r, pl.ds(16, 16)] = b
            return carry2

        lax.fori_loop(0, _ZR, widen, 0)

        @pl.when(c == 0)
        def _():
            pltpu.sync_copy(fbuf, out.at[pl.ds(r0, _ZR), pl.ds(0, 32)])

        @pl.when(c == 1)
        def _():
            pltpu.sync_copy(fbuf, out.at[pl.ds(r0, _ZR), pl.ds(32, 32)])

        return carry

    lax.fori_loop(0, _OPT // _ZR, drain, 0)


def kernel(x_hr, edge_index, edge_type, node_type, W):
    wt = W.reshape(_NET, _AUG, _OUT_C).transpose(1, 0, 2).reshape(_AUG, _YC)
    wt = wt.reshape(_AUG, 14, 2, 16).transpose(0, 1, 3, 2).reshape(_AUG, _YC)
    wp = jnp.zeros((_WROWS, _YC), jnp.float32).at[:_AUG].set(wt / 7.0)
    nt2 = node_type.reshape(_N, 1)
    y = _y_call(x_hr, nt2, wp)
    y2 = y.reshape(_N * 14, 32)
    return _sc_call(y2, edge_index[0], edge_index[1], edge_type)


# speedup vs baseline: 1.0583x; 1.0583x over previous
import functools

import jax
import jax.numpy as jnp
from jax import lax
from jax.experimental import pallas as pl
from jax.experimental.pallas import tpu as pltpu
from jax.experimental.pallas import tpu_sc as plsc

_N = 100000
_E = 1600000
_IN_C = 32
_OUT_C = 64
_NET = 7
_AUG = 37
_YC = _NET * _OUT_C
_WROWS = 40

_ROW_BLK = 2048
_GRID = (_N + _ROW_BLK - 1) // _ROW_BLK

_NTILE = 16
_BLK = 400
_EPT = _E // _NTILE
_BLOCKS = _EPT // _BLK
_PAIRS = _BLOCKS // 2
_ACC_ROWS = _N + 8
_OPT = _N // _NTILE
_ZR = 125


def _y_body(x_ref, nt_ref, w_ref, y_ref):
    xb = x_ref[...]
    ntb = nt_ref[...]
    w1 = w_ref[:_IN_C, :]
    w2 = w_ref[_IN_C:, :]
    oh = (lax.broadcasted_iota(jnp.int32, (_ROW_BLK, _WROWS - _IN_C), 1)
          == ntb).astype(jnp.float32)
    acc = jnp.dot(xb, w1, preferred_element_type=jnp.float32)
    acc = acc + jnp.dot(oh, w2, preferred_element_type=jnp.float32)
    y_ref[...] = acc.astype(jnp.bfloat16)


_y_call = pl.pallas_call(
    _y_body,
    grid=(_GRID,),
    in_specs=[
        pl.BlockSpec((_ROW_BLK, _IN_C), lambda i: (i, 0)),
        pl.BlockSpec((_ROW_BLK, 1), lambda i: (i, 0)),
        pl.BlockSpec((_WROWS, _YC), lambda i: (0, 0)),
    ],
    out_specs=pl.BlockSpec((_ROW_BLK, _YC), lambda i: (i, 0)),
    out_shape=jax.ShapeDtypeStruct((_N, _YC), jnp.bfloat16),
)


@functools.partial(
    pl.kernel,
    out_type=jax.ShapeDtypeStruct((_N, _OUT_C), jnp.float32),
    mesh=plsc.VectorSubcoreMesh(core_axis_name="c", subcore_axis_name="s"),
    compiler_params=pltpu.CompilerParams(use_tc_tiling_on_sc=False,
                                         needs_layout_passes=False),
    scratch_types=[
        pltpu.VMEM((2, 1, _BLK), jnp.int32),
        pltpu.VMEM((2, 1, _BLK), jnp.int32),
        pltpu.VMEM((2, 1, _BLK), jnp.int32),
        pltpu.VMEM((2, 1, _BLK), jnp.int32),
        pltpu.VMEM((2, 1, _BLK, 32), jnp.bfloat16),
        pltpu.VMEM((_ZR, 32), jnp.bfloat16),
        pltpu.VMEM((_ZR, 32), jnp.bfloat16),
        pltpu.VMEM((_ZR, 32), jnp.float32),
        pltpu.VMEM_SHARED((_ACC_ROWS, 32), jnp.bfloat16),
        pltpu.SemaphoreType.DMA,
        pltpu.SemaphoreType.DMA,
    ],
)
def _sc_call(y2, rowm, colm, typm, out, row_v, col_v, typ_v, idx_v, ybuf,
             zbuf, cbuf, fbuf, acc, sem_e, sem_g):
    c = lax.axis_index("c")
    s = lax.axis_index("s")

    def zfill(i, carry):
        zbuf[i, :] = jnp.zeros((32,), jnp.bfloat16)
        return carry

    lax.fori_loop(0, _ZR, zfill, 0)

    def edges_start(e0, p):
        pltpu.async_copy(rowm.at[pl.ds(e0, _BLK)], row_v.at[p, 0], sem_e)
        pltpu.async_copy(colm.at[pl.ds(e0, _BLK)], col_v.at[p, 0], sem_e)
        pltpu.async_copy(typm.at[pl.ds(e0, _BLK)], typ_v.at[p, 0], sem_e)

    def edges_wait(p):
        pltpu.make_async_copy(rowm.at[pl.ds(0, _BLK)], row_v.at[p, 0],
                              sem_e).wait()
        pltpu.make_async_copy(colm.at[pl.ds(0, _BLK)], col_v.at[p, 0],
                              sem_e).wait()
        pltpu.make_async_copy(typm.at[pl.ds(0, _BLK)], typ_v.at[p, 0],
                              sem_e).wait()

    def idx_compute(p):
        for k in range(_BLK // 16):
            sl = pl.ds(k * 16, 16)
            idx_v[p, 0, sl] = (col_v[p, 0, sl] * 14
                               + typ_v[p, 0, sl] * 2 + c)

    def gather_start(p):
        pltpu.async_copy(y2.at[idx_v.at[p, 0]], ybuf.at[p, 0], sem_g)

    def gather_wait(p):
        pltpu.make_async_copy(y2.at[idx_v.at[p, 0]], ybuf.at[p, 0],
                              sem_g).wait()

    def scatter(p):
        pltpu.sync_copy(ybuf.at[p, 0], acc.at[row_v.at[p, 0]], add=True)

    def zcp(z, carry):
        pltpu.sync_copy(zbuf, acc.at[pl.ds(s * _OPT + z * _ZR, _ZR), :])
        return carry

    lax.fori_loop(0, _OPT // _ZR, zcp, 0)
    plsc.subcore_barrier()

    base = s * _EPT

    for k in range(_BLK // 16):
        sl = pl.ds(k * 16, 16)
        idx_v[1, 0, sl] = jnp.zeros((16,), jnp.int32)
        row_v[1, 0, sl] = jnp.full((16,), _N, jnp.int32)
    gather_start(1)
    edges_start(base, 0)

    def pair_body(g, carry):
        b0 = base + 2 * g * _BLK
        edges_wait(0)
        idx_compute(0)
        gather_wait(1)
        gather_start(0)
        scatter(1)
        edges_start(b0 + _BLK, 1)
        edges_wait(1)
        idx_compute(1)
        gather_wait(0)
        gather_start(1)
        scatter(0)
        @pl.when(g < _PAIRS - 1)
        def _():
            edges_start(b0 + 2 * _BLK, 0)
        return carry

    lax.fori_loop(0, _PAIRS, pair_body, 0)
    gather_wait(1)
    scatter(1)
    plsc.subcore_barrier()

    def drain(z, carry):
        r0 = s * _OPT + z * _ZR
        pltpu.sync_copy(acc.at[pl.ds(r0, _ZR), :], cbuf)

        def widen(r, carry2):
            a, b = plsc.unpack(cbuf[r, :], format=plsc.PackFormat.INTERLEAVED)
            fbuf[r, pl.ds(0, 16)] = a
            fbuf[r, pl.ds(16, 16)] = b
            return carry2

        lax.fori_loop(0, _ZR, widen, 0)

        @pl.when(c == 0)
        def _():
            pltpu.sync_copy(fbuf, out.at[pl.ds(r0, _ZR), pl.ds(0, 32)])

        @pl.when(c == 1)
        def _():
            pltpu.sync_copy(fbuf, out.at[pl.ds(r0, _ZR), pl.ds(32, 32)])

        return carry

    lax.fori_loop(0, _OPT // _ZR, drain, 0)


def kernel(x_hr, edge_index, edge_type, node_type, W):
    wt = W.reshape(_NET, _AUG, _OUT_C).transpose(1, 0, 2).reshape(_AUG, _YC)
    wt = wt.reshape(_AUG, 14, 2, 16).transpose(0, 1, 3, 2).reshape(_AUG, _YC)
    wp = jnp.zeros((_WROWS, _YC), jnp.float32).at[:_AUG].set(wt / 7.0)
    nt2 = node_type.reshape(_N, 1)
    y = _y_call(x_hr, nt2, wp)
    y2 = y.reshape(_N * 14, 32)
    return _sc_call(y2, edge_index[0], edge_index[1], edge_type)
